# baseline (speedup 1.0000x reference)
import functools

import jax
import jax.numpy as jnp
from jax import lax
from jax.experimental import pallas as pl
from jax.experimental.pallas import tpu as pltpu
from jax.experimental.pallas import tpu_sc as plsc

_NC = 2
_NS = 16
_LANES = 16
_NW = _NC * _NS
_NBUF = 3


def _pos_encoding(max_len, d_model):
    even_i = jnp.arange(0, d_model, 2).astype(jnp.float32)
    denominator = jnp.power(10000.0, even_i / d_model)
    position = jnp.arange(max_len).reshape(max_len, 1).astype(jnp.float32)
    even_pe = jnp.sin(position / denominator)
    odd_pe = jnp.cos(position / denominator)
    return jnp.stack([even_pe, odd_pe], axis=2).reshape(max_len, d_model)


def _build_sc_call(T, L, B, V, D, per_w):
    pos_per_w = per_w // B
    C = _LANES
    n_h = pos_per_w // C
    n_chunks = B * n_h
    n_vec = D // _LANES
    mesh = plsc.VectorSubcoreMesh(core_axis_name="c", subcore_axis_name="s")

    scratch = [
        pltpu.VMEM((B, pos_per_w), jnp.int32),
        pltpu.VMEM((pos_per_w, D), jnp.float32),
    ]
    scratch += [pltpu.VMEM((C, D), jnp.float32) for _ in range(_NBUF)]
    scratch += [pltpu.SemaphoreType.DMA for _ in range(2 + 2 * _NBUF)]

    @functools.partial(
        pl.kernel,
        mesh=mesh,
        out_type=jax.ShapeDtypeStruct((T, D), jnp.float32),
        scratch_types=scratch,
    )
    def emb_kernel(tok_hbm, table_hbm, pe_hbm, out_hbm, *sc):
        tokb, pev = sc[0], sc[1]
        rows = sc[2:2 + _NBUF]
        tsem, psem = sc[2 + _NBUF], sc[3 + _NBUF]
        gsem = sc[4 + _NBUF:4 + 2 * _NBUF]
        ssem = sc[4 + 2 * _NBUF:4 + 3 * _NBUF]

        w = lax.axis_index("s") * _NC + lax.axis_index("c")
        pe0 = w * pos_per_w

        pe_cp = pltpu.async_copy(pe_hbm.at[pl.ds(pe0, pos_per_w)], pev, psem)
        tok_cp = [
            pltpu.async_copy(
                tok_hbm.at[b, pl.ds(pe0, pos_per_w)], tokb.at[b], tsem
            )
            for b in range(B)
        ]
        for cp in tok_cp:
            cp.wait()
        pe_cp.wait()

        gat = [None] * _NBUF
        sct = [None] * _NBUF

        def start_chunk(j):
            p = j % _NBUF
            b, h = j // n_h, j % n_h
            gat[p] = pltpu.async_copy(
                table_hbm.at[tokb.at[b, pl.ds(h * C, C)]], rows[p], gsem[p]
            )

        def add_chunk(rows_b, h):
            def body(q, carry):
                o = q * _LANES
                for k in range(C):
                    pe_reg = pev[h * C + k, pl.ds(o, _LANES)]
                    plsc.addupdate(rows_b.at[k, pl.ds(o, _LANES)], pe_reg)
                return carry

            lax.fori_loop(0, n_vec, body, 0)

        start_chunk(0)
        start_chunk(1)
        for j in range(n_chunks):
            p = j % _NBUF
            b, h = j // n_h, j % n_h
            gat[p].wait()
            add_chunk(rows[p], h)
            sct[p] = pltpu.async_copy(
                rows[p], out_hbm.at[pl.ds(b * L + pe0 + h * C, C)], ssem[p]
            )
            nxt = j + _NBUF - 1
            if nxt < n_chunks:
                q = nxt % _NBUF
                if sct[q] is not None:
                    sct[q].wait()
                start_chunk(nxt)
        for p in range(_NBUF):
            sct[p].wait()

    return emb_kernel


def kernel(tokens, table):
    B, L = tokens.shape
    V, D = table.shape
    T = B * L
    per_w = T // _NW
    pe = _pos_encoding(L, D)
    emb_kernel = _build_sc_call(T, L, B, V, D, per_w)
    out = emb_kernel(tokens, table, pe)
    return out.reshape(B, L, D)


# speedup vs baseline: 1.2563x; 1.2563x over previous
import functools

import jax
import jax.numpy as jnp
import numpy as np
from jax import lax
from jax.experimental import pallas as pl
from jax.experimental.pallas import tpu as pltpu
from jax.experimental.pallas import tpu_sc as plsc

_NC = 2
_NS = 16
_LANES = 16
_NW = _NC * _NS
_NBUF = 3


def _pos_encoding(max_len, d_model):
    even_i = np.arange(0, d_model, 2, dtype=np.float32)
    denominator = np.power(np.float32(10000.0), even_i / np.float32(d_model))
    position = np.arange(max_len, dtype=np.float32).reshape(max_len, 1)
    even_pe = np.sin(position / denominator, dtype=np.float32)
    odd_pe = np.cos(position / denominator, dtype=np.float32)
    stacked = np.stack([even_pe, odd_pe], axis=2)
    return stacked.reshape(max_len, d_model).astype(np.float32)


def _build_sc_call(T, L, B, V, D, per_w):
    pos_per_w = per_w // B
    C = _LANES
    n_h = pos_per_w // C
    n_chunks = B * n_h
    n_vec = D // _LANES
    mesh = plsc.VectorSubcoreMesh(core_axis_name="c", subcore_axis_name="s")

    scratch = [
        pltpu.VMEM((B, pos_per_w), jnp.int32),
        pltpu.VMEM((pos_per_w, D), jnp.float32),
    ]
    scratch += [pltpu.VMEM((C, D), jnp.float32) for _ in range(_NBUF)]
    scratch += [pltpu.SemaphoreType.DMA for _ in range(2 + 2 * _NBUF)]

    @functools.partial(
        pl.kernel,
        mesh=mesh,
        out_type=jax.ShapeDtypeStruct((T, D), jnp.float32),
        scratch_types=scratch,
    )
    def emb_kernel(tok_hbm, table_hbm, pe_hbm, out_hbm, *sc):
        tokb, pev = sc[0], sc[1]
        rows = sc[2:2 + _NBUF]
        tsem, psem = sc[2 + _NBUF], sc[3 + _NBUF]
        gsem = sc[4 + _NBUF:4 + 2 * _NBUF]
        ssem = sc[4 + 2 * _NBUF:4 + 3 * _NBUF]

        w = lax.axis_index("s") * _NC + lax.axis_index("c")
        pe0 = w * pos_per_w

        pe_cp = pltpu.async_copy(pe_hbm.at[pl.ds(pe0, pos_per_w)], pev, psem)
        tok_cp = [
            pltpu.async_copy(
                tok_hbm.at[b, pl.ds(pe0, pos_per_w)], tokb.at[b], tsem
            )
            for b in range(B)
        ]
        for cp in tok_cp:
            cp.wait()
        pe_cp.wait()

        gat = [None] * _NBUF
        sct = [None] * _NBUF

        def start_chunk(j):
            p = j % _NBUF
            b, h = j // n_h, j % n_h
            gat[p] = pltpu.async_copy(
                table_hbm.at[tokb.at[b, pl.ds(h * C, C)]], rows[p], gsem[p]
            )

        def add_chunk(rows_b, h):
            def body(q, carry):
                o = q * _LANES
                for k in range(C):
                    pe_reg = pev[h * C + k, pl.ds(o, _LANES)]
                    plsc.addupdate(rows_b.at[k, pl.ds(o, _LANES)], pe_reg)
                return carry

            lax.fori_loop(0, n_vec, body, 0)

        start_chunk(0)
        start_chunk(1)
        for j in range(n_chunks):
            p = j % _NBUF
            b, h = j // n_h, j % n_h
            gat[p].wait()
            add_chunk(rows[p], h)
            sct[p] = pltpu.async_copy(
                rows[p], out_hbm.at[pl.ds(b * L + pe0 + h * C, C)], ssem[p]
            )
            nxt = j + _NBUF - 1
            if nxt < n_chunks:
                q = nxt % _NBUF
                if sct[q] is not None:
                    sct[q].wait()
                start_chunk(nxt)
        for p in range(_NBUF):
            sct[p].wait()

    return emb_kernel


def kernel(tokens, table):
    B, L = tokens.shape
    V, D = table.shape
    T = B * L
    per_w = T // _NW
    pe = _pos_encoding(L, D)
    emb_kernel = _build_sc_call(T, L, B, V, D, per_w)
    out = emb_kernel(tokens, table, pe)
    return out.reshape(B, L, D)


# speedup vs baseline: 1.5954x; 1.2699x over previous
import functools

import jax
import jax.numpy as jnp
import numpy as np
from jax import lax
from jax.experimental import pallas as pl
from jax.experimental.pallas import tpu as pltpu
from jax.experimental.pallas import tpu_sc as plsc

_NC = 2
_NS = 16
_LANES = 16
_NW = _NC * _NS
_NBUF = 3


def _pos_encoding(max_len, d_model):
    even_i = np.arange(0, d_model, 2, dtype=np.float32)
    denominator = np.power(np.float32(10000.0), even_i / np.float32(d_model))
    position = np.arange(max_len, dtype=np.float32).reshape(max_len, 1)
    even_pe = np.sin(position / denominator, dtype=np.float32)
    odd_pe = np.cos(position / denominator, dtype=np.float32)
    stacked = np.stack([even_pe, odd_pe], axis=2)
    return stacked.reshape(max_len, d_model).astype(np.float32)


def _build_sc_call(T, L, B, V, D, per_w, C):
    n_chunks = per_w // C
    ppc = C // B
    pos_per_w = per_w // B
    n_vec = D // _LANES
    mesh = plsc.VectorSubcoreMesh(core_axis_name="c", subcore_axis_name="s")

    scratch = [
        pltpu.VMEM((n_chunks, C), jnp.int32),
    ]
    scratch += [pltpu.VMEM((C,), jnp.int32) for _ in range(_NBUF)]
    scratch += [pltpu.VMEM((C, D), jnp.float32) for _ in range(_NBUF)]
    scratch += [pltpu.VMEM((ppc, D), jnp.float32) for _ in range(_NBUF)]
    scratch += [pltpu.SemaphoreType.DMA for _ in range(4 * _NBUF)]

    @functools.partial(
        pl.kernel,
        mesh=mesh,
        out_type=jax.ShapeDtypeStruct((T, D), jnp.float32),
        scratch_types=scratch,
    )
    def emb_kernel(tok_hbm, table_hbm, pe_hbm, out_hbm, *sc):
        oidx_v = sc[0]
        cidx = sc[1:1 + _NBUF]
        rows = sc[1 + _NBUF:1 + 2 * _NBUF]
        peb = sc[1 + 2 * _NBUF:1 + 3 * _NBUF]
        csem = sc[1 + 3 * _NBUF:1 + 4 * _NBUF]
        gsem = sc[1 + 4 * _NBUF:1 + 5 * _NBUF]
        psem = sc[1 + 5 * _NBUF:1 + 6 * _NBUF]
        ssem = sc[1 + 6 * _NBUF:1 + 7 * _NBUF]

        w = lax.axis_index("s") * _NC + lax.axis_index("c")
        pe0 = w * pos_per_w
        lane = jnp.arange(_LANES, dtype=jnp.int32)

        def build_body(p, carry):
            oidx_v[p // ppc, pl.ds(lax.rem(p, ppc) * B, B)] = (
                lane * L + pe0 + p
            )
            return carry

        lax.fori_loop(0, pos_per_w, build_body, 0)

        fetch = [None] * _NBUF
        gat = [None] * _NBUF
        pes = [None] * _NBUF
        sct = [None] * _NBUF

        def start_fetch(j):
            p = j % _NBUF
            fetch[p] = pltpu.async_copy(
                tok_hbm.at[oidx_v.at[j]], cidx[p], csem[p]
            )

        def start_gather(j):
            p = j % _NBUF
            gat[p] = pltpu.async_copy(
                table_hbm.at[cidx[p]], rows[p], gsem[p]
            )
            pes[p] = pltpu.async_copy(
                pe_hbm.at[pl.ds(pe0 + j * ppc, ppc)], peb[p], psem[p]
            )

        def add_chunk(rows_b, pe_b):
            def body(q, carry):
                o = q * _LANES
                for t in range(ppc):
                    pe_reg = pe_b[t, pl.ds(o, _LANES)]
                    for b in range(B):
                        plsc.addupdate(
                            rows_b.at[t * B + b, pl.ds(o, _LANES)], pe_reg
                        )
                return carry

            lax.fori_loop(0, n_vec, body, 0)

        for j in range(_NBUF):
            start_fetch(j)
        for j in range(2):
            fetch[j].wait()
            fetch[j] = None
            start_gather(j)

        for j in range(n_chunks):
            p = j % _NBUF
            gat[p].wait()
            pes[p].wait()
            add_chunk(rows[p], peb[p])
            sct[p] = pltpu.async_copy(rows[p], out_hbm.at[oidx_v.at[j]], ssem[p])
            if j + _NBUF < n_chunks:
                start_fetch(j + _NBUF)
            nxt = j + _NBUF - 1
            if nxt < n_chunks:
                q = nxt % _NBUF
                if sct[q] is not None:
                    sct[q].wait()
                if fetch[q] is not None:
                    fetch[q].wait()
                    fetch[q] = None
                start_gather(nxt)
        for p in range(_NBUF):
            sct[p].wait()

    return emb_kernel


def kernel(tokens, table):
    B, L = tokens.shape
    V, D = table.shape
    T = B * L
    per_w = T // _NW
    C = 32
    pe = _pos_encoding(L, D)
    emb_kernel = _build_sc_call(T, L, B, V, D, per_w, C)
    out = emb_kernel(tokens.reshape(T), table, pe)
    return out.reshape(B, L, D)
